# baseline (speedup 1.0000x reference)
import functools

import jax
import jax.numpy as jnp
from jax import lax
from jax.experimental import pallas as pl
from jax.experimental.pallas import tpu as pltpu
from jax.experimental.pallas import tpu_sc as plsc

_H = 256
_X = 512
_Y = 512
_FILL = 16
_L = 16
_NC = 2
_NS = 16
_NW = _NC * _NS
_CH = (_X // _NW) * _Y
_NCHUNK = _CH // _L
_ZB = 4
_NGRP = _H // (2 * _ZB)


def _sc_body(hm_hbm, sm_hbm, tab_hbm, out_hbm,
             tab_v, hm_v, sm_v, h_v, sem_v, p0, p1, dsem0, dsem1):
    wid = lax.axis_index("s") * _NC + lax.axis_index("c")
    base = wid * _CH
    pltpu.sync_copy(hm_hbm.at[pl.ds(base, _CH)], hm_v)
    pltpu.sync_copy(sm_hbm.at[pl.ds(base, _CH)], sm_v)
    pltpu.sync_copy(tab_hbm, tab_v)
    tab = tab_v[...]

    @plsc.parallel_loop(0, _NCHUNK, unroll=8)
    def _prep(i):
        s = pl.ds(i * _L, _L)
        h_v[s] = jnp.clip((hm_v[s] * float(_H - 1)).astype(jnp.int32),
                          0, _H - 1)
        sem_v[s] = tab.at[sm_v[s]].get(mode="promise_in_bounds")

    def _compute(z0, pbuf):
        @plsc.parallel_loop(0, _NCHUNK, unroll=4)
        def _chunk(i):
            s = pl.ds(i * _L, _L)
            r = i >> 5
            c = (i & 31) * _L
            h = h_v[s]
            sem = sem_v[s]
            d0 = z0 - h
            for dz in range(_ZB):
                pbuf[dz, r, pl.ds(c, _L)] = sem

    rows = _X // _NW
    row0 = wid * rows

    def _start(z0, pbuf, dsem):
        return pltpu.async_copy(
            pbuf, out_hbm.at[pl.ds(z0, _ZB), pl.ds(row0, rows)], dsem)

    def _drain(pbuf, dsem):
        pltpu.make_async_copy(
            pbuf, out_hbm.at[pl.ds(0, _ZB), pl.ds(row0, rows)], dsem).wait()

    def _outer(g, _):
        z0 = g * (2 * _ZB)

        @pl.when(g > 0)
        def _w0():
            _drain(p0, dsem0)
        _compute(z0, p0)
        _start(z0, p0, dsem0)

        @pl.when(g > 0)
        def _w1():
            _drain(p1, dsem1)
        _compute(z0 + _ZB, p1)
        _start(z0 + _ZB, p1, dsem1)
        return _

    lax.fori_loop(0, _NGRP, _outer, 0)
    _drain(p0, dsem0)
    _drain(p1, dsem1)


@functools.partial(
    pl.kernel,
    out_type=jax.ShapeDtypeStruct((_H, _X, _Y), jnp.float32),
    mesh=plsc.VectorSubcoreMesh(core_axis_name="c", subcore_axis_name="s"),
    scratch_types=[
        pltpu.VMEM((_L,), jnp.float32),
        pltpu.VMEM((_CH,), jnp.float32),
        pltpu.VMEM((_CH,), jnp.int32),
        pltpu.VMEM((_CH,), jnp.int32),
        pltpu.VMEM((_CH,), jnp.float32),
        pltpu.VMEM((_ZB, _X // _NW, _Y), jnp.float32),
        pltpu.VMEM((_ZB, _X // _NW, _Y), jnp.float32),
        pltpu.SemaphoreType.DMA,
        pltpu.SemaphoreType.DMA,
    ],
)
def _sc_fill(hm, sm, tab, out, *rest):
    _sc_body(hm, sm, tab, out, *rest)


def kernel(height_map, semantic_map, biome2mclabels):
    hm = height_map.reshape(_X * _Y)
    sm = semantic_map.reshape(_X * _Y)
    tab = jnp.zeros((_L,), jnp.float32).at[:10].set(biome2mclabels)
    return _sc_fill(hm, sm, tab)


# speedup vs baseline: 1.0048x; 1.0048x over previous
import functools

import jax
import jax.numpy as jnp
from jax import lax
from jax.experimental import pallas as pl
from jax.experimental.pallas import tpu as pltpu
from jax.experimental.pallas import tpu_sc as plsc

_H = 256
_X = 512
_Y = 512
_FILL = 16
_L = 16
_NC = 2
_NS = 16
_NW = _NC * _NS
_CH = (_X // _NW) * _Y
_NCHUNK = _CH // _L
_ZB = 4
_NGRP = _H // (2 * _ZB)


def _sc_body(hm_hbm, sm_hbm, tab_hbm, out_hbm,
             tab_v, hm_v, sm_v, h_v, sem_v, p0, p1, dsem0, dsem1):
    wid = lax.axis_index("s") * _NC + lax.axis_index("c")
    base = wid * _CH
    pltpu.sync_copy(hm_hbm.at[pl.ds(base, _CH)], hm_v)
    pltpu.sync_copy(sm_hbm.at[pl.ds(base, _CH)], sm_v)
    pltpu.sync_copy(tab_hbm, tab_v)
    tab = tab_v[...]

    @plsc.parallel_loop(0, _NCHUNK, unroll=8)
    def _prep(i):
        s = pl.ds(i * _L, _L)
        h_v[s] = jnp.clip((hm_v[s] * float(_H - 1)).astype(jnp.int32),
                          0, _H - 1)
        sem_v[s] = tab.at[sm_v[s]].get(mode="promise_in_bounds")

    def _compute(z0, pbuf):
        @plsc.parallel_loop(0, _NCHUNK, unroll=4)
        def _chunk(i):
            s = pl.ds(i * _L, _L)
            r = i >> 5
            c = (i & 31) * _L
            h = h_v[s]
            sem = sem_v[s]
            d0 = z0 - h
            for dz in range(_ZB):
                d = (d0 + dz).astype(jnp.uint32)
                pbuf[dz, r, pl.ds(c, _L)] = jnp.where(d <= _FILL, sem, 0.0)

    rows = _X // _NW
    row0 = wid * rows

    def _start(z0, pbuf, dsem):
        return pltpu.async_copy(
            pbuf, out_hbm.at[pl.ds(z0, _ZB), pl.ds(row0, rows)], dsem)

    def _drain(pbuf, dsem):
        pltpu.make_async_copy(
            pbuf, out_hbm.at[pl.ds(0, _ZB), pl.ds(row0, rows)], dsem).wait()

    def _outer(g, _):
        z0 = g * (2 * _ZB)

        @pl.when(g > 0)
        def _w0():
            _drain(p0, dsem0)
        _compute(z0, p0)
        _start(z0, p0, dsem0)

        @pl.when(g > 0)
        def _w1():
            _drain(p1, dsem1)
        _compute(z0 + _ZB, p1)
        _start(z0 + _ZB, p1, dsem1)
        return _

    lax.fori_loop(0, _NGRP, _outer, 0)
    _drain(p0, dsem0)
    _drain(p1, dsem1)


@functools.partial(
    pl.kernel,
    out_type=jax.ShapeDtypeStruct((_H, _X, _Y), jnp.float32),
    mesh=plsc.VectorSubcoreMesh(core_axis_name="c", subcore_axis_name="s"),
    scratch_types=[
        pltpu.VMEM((_L,), jnp.float32),
        pltpu.VMEM((_CH,), jnp.float32),
        pltpu.VMEM((_CH,), jnp.int32),
        pltpu.VMEM((_CH,), jnp.int32),
        pltpu.VMEM((_CH,), jnp.float32),
        pltpu.VMEM((_ZB, _X // _NW, _Y), jnp.float32),
        pltpu.VMEM((_ZB, _X // _NW, _Y), jnp.float32),
        pltpu.SemaphoreType.DMA,
        pltpu.SemaphoreType.DMA,
    ],
)
def _sc_fill(hm, sm, tab, out, *rest):
    _sc_body(hm, sm, tab, out, *rest)


def kernel(height_map, semantic_map, biome2mclabels):
    hm = height_map.reshape(_X * _Y)
    sm = semantic_map.reshape(_X * _Y)
    tab = jnp.zeros((_L,), jnp.float32).at[:10].set(biome2mclabels)
    return _sc_fill(hm, sm, tab)
